# baseline (speedup 1.0000x reference)
import jax
import jax.numpy as jnp
from jax import lax
from jax.experimental import pallas as pl
from jax.experimental.pallas import tpu as pltpu
from jax.experimental.pallas import tpu_sc as plsc

B, C, F, H, W = 8, 3, 5, 224, 224
HIN, WIN = H + F - 1, W + F - 1
NC, NS, L = 2, 16, 16
RB = 28
RBW = RB * W
NQ = H // RB
NTASK = B * C * NQ
TPW = NTASK // (NC * NS)
GX = W // L
K = F * F

PADW = 240
PADH = 234
PADN = PADH * PADW
PX, PY = 8, 2
IDXC = (PY - 4) * PADW + (PX - 4)
BLO, BHI = 2.0, 232.9


def _body(in_ref, vert_ref, horiz_ref, offx_ref, offy_ref, mask_ref,
          out_ref, img, acc,
          offx_a, offy_a, msk_a, vert_a, horiz_a,
          offx_b, offy_b, msk_b, vert_b, horiz_b,
          sem_a, sem_b):
  wid = lax.axis_index("s") * NC + lax.axis_index("c")
  xiota = lax.broadcasted_iota(jnp.int32, (L,), 0).astype(jnp.float32)

  bufs = ((offx_a, offy_a, msk_a, vert_a, horiz_a, sem_a),
          (offx_b, offy_b, msk_b, vert_b, horiz_b, sem_b))

  def issue(r, b, r0w, par):
    r = jnp.minimum(r, RB - 1)
    rw = r0w + r * W
    ox, oy, mk, vt, hz, sem = bufs[par]
    pltpu.async_copy(offx_ref.at[b, :, pl.ds(rw, W)], ox, sem)
    pltpu.async_copy(offy_ref.at[b, :, pl.ds(rw, W)], oy, sem)
    pltpu.async_copy(mask_ref.at[b, :, pl.ds(rw, W)], mk, sem)
    pltpu.async_copy(vert_ref.at[b, :, pl.ds(rw, W)], vt, sem)
    pltpu.async_copy(horiz_ref.at[b, :, pl.ds(rw, W)], hz, sem)

  def drain(par):
    ox, oy, mk, vt, hz, sem = bufs[par]
    pltpu.make_async_copy(offx_ref.at[0, :, pl.ds(0, W)], ox, sem).wait()
    pltpu.make_async_copy(offy_ref.at[0, :, pl.ds(0, W)], oy, sem).wait()
    pltpu.make_async_copy(mask_ref.at[0, :, pl.ds(0, W)], mk, sem).wait()
    pltpu.make_async_copy(vert_ref.at[0, :, pl.ds(0, W)], vt, sem).wait()
    pltpu.make_async_copy(horiz_ref.at[0, :, pl.ds(0, W)], hz, sem).wait()

  def compute_row(r, r0, par):
    ox, oy, mk, vt, hz, _ = bufs[par]
    ybase_r = r0 + r + 4

    @plsc.parallel_loop(0, GX, 1, unroll=1)
    def g_loop(g):
      sl = pl.ds(g * L, L)
      xg = (g * L).astype(jnp.float32) + xiota
      xgs = [xg + (fx + 4.0) for fx in range(F)]

      def fy_loop(fy, accv):
        ybase = (ybase_r + fy).astype(jnp.float32)
        vtv = vt[fy, sl]
        part = None
        for fx in range(F):
          k = fy * F + fx
          posy = oy[k, sl] + ybase
          posx = ox[k, sl] + xgs[fx]
          posy = jnp.clip(posy, BLO, BHI)
          posx = jnp.clip(posx, BLO, BHI)
          ty = posy.astype(jnp.int32)
          tx = posx.astype(jnp.int32)
          ay = posy - ty.astype(jnp.float32)
          ax = posx - tx.astype(jnp.float32)
          f00 = ty * PADW + (tx + IDXC)
          f10 = f00 + PADW
          g00 = plsc.load_gather(img, [f00])
          g01 = plsc.load_gather(img, [f00 + 1])
          g10 = plsc.load_gather(img, [f10])
          g11 = plsc.load_gather(img, [f10 + 1])
          by = 1.0 - ay
          bx = 1.0 - ax
          samp = by * (bx * g00 + ax * g01) + ay * (bx * g10 + ax * g11)
          wsep = vtv * hz[fx, sl] * mk[k, sl]
          contrib = samp * wsep
          part = contrib if part is None else part + contrib
        return accv + part

      accv = lax.fori_loop(0, F, fy_loop, jnp.zeros((L,), jnp.float32))
      acc[pl.ds(r * W + g * L, L)] = accv

  def task_loop(ti, carry):
    t = wid * TPW + ti
    in_main = wid < 24
    bc = jnp.where(in_main, wid, 3 * (wid - 24) + ti // 2)
    q = jnp.where(in_main, ti, 6 + ti % 2)
    b = bc // C
    c = bc % C
    r0 = q * RB
    r0w = q * RBW

    @pl.when(jnp.logical_or(ti == 0, jnp.logical_and(~in_main, ti % 2 == 0)))
    def _():
      pltpu.sync_copy(in_ref.at[b, c], img)

    issue(jnp.int32(0), b, r0w, 0)
    issue(jnp.int32(1), b, r0w, 1)

    def rr_loop(rr, _):
      r = 2 * rr
      drain(0)
      compute_row(r, r0, 0)
      issue(r + 2, b, r0w, 0)
      drain(1)
      compute_row(r + 1, r0, 1)
      issue(r + 3, b, r0w, 1)
      return 0
    lax.fori_loop(0, RB // 2, rr_loop, 0)
    drain(0)
    drain(1)

    pltpu.sync_copy(acc, out_ref.at[b, c, pl.ds(r0w, RBW)])
    return 0

  lax.fori_loop(0, TPW, task_loop, 0)


@jax.jit
def kernel(input, vertical, horizontal, offset_x, offset_y, mask):
  inp = jnp.pad(input, ((0, 0), (0, 0),
                        (PY, PADH - HIN - PY),
                        (PX, PADW - WIN - PX))).reshape(B, C, PADN)
  mesh = plsc.VectorSubcoreMesh(
      core_axis_name="c", subcore_axis_name="s",
      num_cores=NC, num_subcores=NS)
  rowk = pltpu.VMEM((K, W), jnp.float32)
  rowf = pltpu.VMEM((F, W), jnp.float32)
  f = pl.kernel(
      _body,
      out_type=jax.ShapeDtypeStruct((B, C, H * W), jnp.float32),
      mesh=mesh,
      compiler_params=pltpu.CompilerParams(
          use_tc_tiling_on_sc=False, needs_layout_passes=False),
      scratch_types=[
          pltpu.VMEM((PADN,), jnp.float32),
          pltpu.VMEM((RBW,), jnp.float32),
          rowk, rowk, rowk, rowf, rowf,
          rowk, rowk, rowk, rowf, rowf,
          pltpu.SemaphoreType.DMA,
          pltpu.SemaphoreType.DMA,
      ],
  )
  out = f(inp, vertical.reshape(B, F, H * W), horizontal.reshape(B, F, H * W),
          offset_x.reshape(B, K, H * W), offset_y.reshape(B, K, H * W),
          mask.reshape(B, K, H * W))
  return out.reshape(B, C, H, W)


# speedup vs baseline: 1.0205x; 1.0205x over previous
import jax
import jax.numpy as jnp
from jax import lax
from jax.experimental import pallas as pl
from jax.experimental.pallas import tpu as pltpu
from jax.experimental.pallas import tpu_sc as plsc

B, C, F, H, W = 8, 3, 5, 224, 224
HIN, WIN = H + F - 1, W + F - 1
NC, NS, L = 2, 16, 16
RB = 28
RBW = RB * W
NQ = H // RB
NTASK = B * C * NQ
TPW = NTASK // (NC * NS)
GX = W // L
NG = RB * GX
K = F * F

PADW = 240
PADH = 234
PADN = PADH * PADW
PX, PY = 8, 2
IDXC = (PY - 4) * PADW + (PX - 4)
BLO, BHI = 2.0, 232.9


def _body(in_ref, vert_ref, horiz_ref, offx_ref, offy_ref, mask_ref,
          out_ref, img, acc,
          offx_a, offy_a, msk_a, vert_a, horiz_a,
          offx_b, offy_b, msk_b, vert_b, horiz_b,
          sem_a, sem_b):
  wid = lax.axis_index("s") * NC + lax.axis_index("c")
  xiota = lax.broadcasted_iota(jnp.int32, (L,), 0).astype(jnp.float32)
  fzero = jnp.zeros((L,), jnp.float32)

  bufs = ((offx_a, offy_a, msk_a, vert_a, horiz_a, sem_a),
          (offx_b, offy_b, msk_b, vert_b, horiz_b, sem_b))

  def issue(k, b, r0w, par):
    k = jnp.minimum(k, K - 1)
    fy = k // F
    fx = k % F
    ox, oy, mk, vt, hz, sem = bufs[par]
    pltpu.async_copy(offx_ref.at[b, k, pl.ds(r0w, RBW)], ox, sem)
    pltpu.async_copy(offy_ref.at[b, k, pl.ds(r0w, RBW)], oy, sem)
    pltpu.async_copy(mask_ref.at[b, k, pl.ds(r0w, RBW)], mk, sem)
    pltpu.async_copy(vert_ref.at[b, fy, pl.ds(r0w, RBW)], vt, sem)
    pltpu.async_copy(horiz_ref.at[b, fx, pl.ds(r0w, RBW)], hz, sem)

  def drain(par):
    ox, oy, mk, vt, hz, sem = bufs[par]
    pltpu.make_async_copy(offx_ref.at[0, 0, pl.ds(0, RBW)], ox, sem).wait()
    pltpu.make_async_copy(offy_ref.at[0, 0, pl.ds(0, RBW)], oy, sem).wait()
    pltpu.make_async_copy(mask_ref.at[0, 0, pl.ds(0, RBW)], mk, sem).wait()
    pltpu.make_async_copy(vert_ref.at[0, 0, pl.ds(0, RBW)], vt, sem).wait()
    pltpu.make_async_copy(horiz_ref.at[0, 0, pl.ds(0, RBW)], hz, sem).wait()

  def compute(k, r0, par, first=False):
    ox, oy, mk, vt, hz, _ = bufs[par]
    fy = k // F
    ybase0 = r0 + fy + 4
    fxf = (k % F + 4).astype(jnp.float32)

    @plsc.parallel_loop(0, NG, 1, unroll=2)
    def g_loop(i):
      sl = pl.ds(i * L, L)
      ybase = (ybase0 + i // GX).astype(jnp.float32)
      xb = ((i % GX) * L).astype(jnp.float32) + fxf
      posy = oy[sl] + ybase
      posx = (ox[sl] + xb) + xiota
      posy = jnp.clip(posy, BLO, BHI)
      posx = jnp.clip(posx, BLO, BHI)
      ty = posy.astype(jnp.int32)
      tx = posx.astype(jnp.int32)
      ay = posy - ty.astype(jnp.float32)
      ax = posx - tx.astype(jnp.float32)
      f00 = ty * PADW + (tx + IDXC)
      f10 = f00 + PADW
      g00 = plsc.load_gather(img, [f00])
      g01 = plsc.load_gather(img, [f00 + 1])
      g10 = plsc.load_gather(img, [f10])
      g11 = plsc.load_gather(img, [f10 + 1])
      by = 1.0 - ay
      bx = 1.0 - ax
      samp = by * (bx * g00 + ax * g01) + ay * (bx * g10 + ax * g11)
      wsep = vt[sl] * hz[sl] * mk[sl]
      if first:
        acc[sl] = samp * wsep
      else:
        acc[sl] = acc[sl] + samp * wsep

  def task_loop(ti, carry):
    in_main = wid < 24
    bc = jnp.where(in_main, wid, 3 * (wid - 24) + ti // 2)
    q = jnp.where(in_main, ti, 6 + ti % 2)
    b = bc // C
    c = bc % C
    r0 = q * RB
    r0w = q * RBW

    issue(jnp.int32(0), b, r0w, 0)
    issue(jnp.int32(1), b, r0w, 1)

    @pl.when(jnp.logical_or(ti == 0, jnp.logical_and(~in_main, ti % 2 == 0)))
    def _():
      pltpu.sync_copy(in_ref.at[b, c], img)

    drain(0)
    compute(jnp.int32(0), r0, 0, first=True)
    issue(jnp.int32(2), b, r0w, 0)

    def kk_loop(kk, _):
      k = 2 * kk + 1
      drain(1)
      compute(k, r0, 1)
      issue(k + 2, b, r0w, 1)
      drain(0)
      compute(k + 1, r0, 0)
      issue(k + 3, b, r0w, 0)
      return 0
    lax.fori_loop(0, (K - 1) // 2, kk_loop, 0)

    drain(1)
    drain(0)

    pltpu.sync_copy(acc, out_ref.at[b, c, pl.ds(r0w, RBW)])
    return 0

  lax.fori_loop(0, TPW, task_loop, 0)


@jax.jit
def kernel(input, vertical, horizontal, offset_x, offset_y, mask):
  inp = jnp.pad(input, ((0, 0), (0, 0),
                        (PY, PADH - HIN - PY),
                        (PX, PADW - WIN - PX))).reshape(B, C, PADN)
  mesh = plsc.VectorSubcoreMesh(
      core_axis_name="c", subcore_axis_name="s",
      num_cores=NC, num_subcores=NS)
  blk = pltpu.VMEM((RBW,), jnp.float32)
  f = pl.kernel(
      _body,
      out_type=jax.ShapeDtypeStruct((B, C, H * W), jnp.float32),
      mesh=mesh,
      compiler_params=pltpu.CompilerParams(
          use_tc_tiling_on_sc=False, needs_layout_passes=False),
      scratch_types=[
          pltpu.VMEM((PADN,), jnp.float32),
          blk,
          blk, blk, blk, blk, blk,
          blk, blk, blk, blk, blk,
          pltpu.SemaphoreType.DMA,
          pltpu.SemaphoreType.DMA,
      ],
  )
  out = f(inp, vertical.reshape(B, F, H * W), horizontal.reshape(B, F, H * W),
          offset_x.reshape(B, K, H * W), offset_y.reshape(B, K, H * W),
          mask.reshape(B, K, H * W))
  return out.reshape(B, C, H, W)
